# baseline (speedup 1.0000x reference)
import functools

import jax
import jax.numpy as jnp
from jax import lax
from jax.experimental import pallas as pl
from jax.experimental.pallas import tpu as pltpu
from jax.experimental.pallas import tpu_sc as plsc

_B, _N, _M, _D, _K = 1024, 26, 26, 128, 10
_BB = 8
_L = _BB * _N

_NC, _NS = 2, 16
_NW = _NC * _NS
_BPW = _B // _NW


def _dot(a, b):
    return jax.lax.dot(a, b, preferred_element_type=jnp.float32)


def _scores_kernel(f_ref, c_ref, wq_ref, wk_ref, sel_ref, idx_ref):
    wqT = wq_ref[...].T
    wkT = wk_ref[...].T
    f = jnp.concatenate([f_ref[b] for b in range(_BB)], axis=0)
    c = jnp.concatenate([c_ref[b] for b in range(_BB)], axis=0)

    q = _dot(f, wqT)
    k = _dot(c, wkT)
    W = jax.lax.dot_general(
        q, k, (((1,), (1,)), ((), ())),
        preferred_element_type=jnp.float32)

    S = jnp.concatenate(
        [W[_N * b:_N * (b + 1), _M * b:_M * (b + 1)] for b in range(_BB)],
        axis=0)

    vals = []
    wcur = S
    for _ in range(_K):
        mx = jnp.max(wcur, axis=1, keepdims=True)
        vals.append(mx)
        wcur = jnp.where(wcur == mx, -jnp.inf, wcur)

    exps = [jnp.exp(v - vals[0]) for v in vals]
    inv = 1.0 / sum(exps)

    row_iota = jax.lax.broadcasted_iota(jnp.int32, (_L, _L), 0)
    l_iota = jax.lax.broadcasted_iota(jnp.int32, (_L, _L), 1)
    Wm = jnp.where(row_iota // _N == l_iota // _M, W, -jnp.inf)
    l_iota_f = l_iota.astype(jnp.float32)
    goff = pl.program_id(0) * _L
    for kk in range(_K):
        hit = Wm == vals[kk]
        sel_ref[:, pl.ds(kk, 1)] = exps[kk] * inv
        col = jnp.sum(jnp.where(hit, l_iota_f, 0.0), axis=1, keepdims=True)
        idx_ref[:, pl.ds(kk, 1)] = col.astype(jnp.int32) + goff


def _tc_scores(featureVec, contextVec, Wq, Wk):
    return pl.pallas_call(
        _scores_kernel,
        grid=(_B // _BB,),
        in_specs=[
            pl.BlockSpec((_BB, _N, _D), lambda i: (i, 0, 0)),
            pl.BlockSpec((_BB, _M, _D), lambda i: (i, 0, 0)),
            pl.BlockSpec((_D, _D), lambda i: (0, 0)),
            pl.BlockSpec((_D, _D), lambda i: (0, 0)),
        ],
        out_specs=[
            pl.BlockSpec((_L, _K), lambda i: (i, 0)),
            pl.BlockSpec((_L, _K), lambda i: (i, 0)),
        ],
        out_shape=[
            jax.ShapeDtypeStruct((_B * _N, _K), jnp.float32),
            jax.ShapeDtypeStruct((_B * _N, _K), jnp.int32),
        ],
    )(featureVec, contextVec, Wq, Wk)


def _sc_assemble(f3, c2, selr, idxr):
    mesh = plsc.VectorSubcoreMesh(core_axis_name="c", subcore_axis_name="s")
    chunks = [(0, 128), (128, 128), (256, 4)]

    @functools.partial(
        pl.kernel, mesh=mesh,
        out_type=jax.ShapeDtypeStruct((_B, _N * _K, _D), jnp.float32),
        scratch_types=[
            pltpu.VMEM((_N * _K,), jnp.int32),
            pltpu.VMEM((_N * _K, 1), jnp.float32),
            pltpu.VMEM((_N, _D), jnp.float32),
            pltpu.VMEM((_N * _K, _D), jnp.float32),
            pltpu.VMEM((_N * _K, _D), jnp.float32),
            pltpu.SemaphoreType.DMA,
        ],
    )
    def k(f_hbm, c_hbm, sel_hbm, idx_hbm, out_hbm,
          idx_v, sel_v, f_v, rows_v, out_v, sem):
        wid = lax.axis_index("s") * _NC + lax.axis_index("c")

        @pl.loop(0, _BPW)
        def _(j):
            b = wid * _BPW + j
            pltpu.sync_copy(idx_hbm.at[b], idx_v)
            pltpu.sync_copy(sel_hbm.at[b], sel_v)
            pltpu.sync_copy(f_hbm.at[b], f_v)
            copies = [
                pltpu.async_copy(
                    c_hbm.at[idx_v.at[pl.ds(off, sz)]],
                    rows_v.at[pl.ds(off, sz)], sem)
                for off, sz in chunks
            ]
            for cp in copies:
                cp.wait()

            @pl.loop(0, _N)
            def _(n):
                ro = rows_v.at[pl.ds(n * _K, _K)][...]
                se = sel_v.at[pl.ds(n * _K, _K)][...]
                fr = f_v.at[pl.ds(n, 1)][...]
                out_v.at[pl.ds(n * _K, _K)][...] = ro * se * fr

            pltpu.sync_copy(out_v, out_hbm.at[b])

    return k(f3, c2, selr, idxr)


def kernel(featureVec, contextVec, Wq, Wk):
    sel2, idx2 = _tc_scores(featureVec, contextVec, Wq, Wk)
    c2 = contextVec.reshape(_B * _M, _D)
    selr = sel2.reshape(_B, _N * _K, 1)
    idxr = idx2.reshape(_B, _N * _K)
    return _sc_assemble(featureVec, c2, selr, idxr)


# speedup vs baseline: 2.5884x; 2.5884x over previous
import jax
import jax.numpy as jnp
from jax.experimental import pallas as pl

_B, _N, _M, _D, _K = 1024, 26, 26, 128, 10
_BB = 16
_L = _BB * _N


def _dot(a, b):
    return jax.lax.dot(a, b, preferred_element_type=jnp.float32)


def _attn_kernel(f_ref, c_ref, wq_ref, wk_ref, out_ref):
    wqT = wq_ref[...].T
    wkT = wk_ref[...].T
    f = jnp.concatenate([f_ref[b] for b in range(_BB)], axis=0)
    c = jnp.concatenate([c_ref[b] for b in range(_BB)], axis=0)

    q = _dot(f, wqT)
    k = _dot(c, wkT)
    W = jax.lax.dot_general(
        q, k, (((1,), (1,)), ((), ())),
        preferred_element_type=jnp.float32)

    S = jnp.concatenate(
        [W[_N * b:_N * (b + 1), _M * b:_M * (b + 1)] for b in range(_BB)],
        axis=0)

    vals = []
    wcur = S
    for _ in range(_K):
        mx = jnp.max(wcur, axis=1, keepdims=True)
        vals.append(mx)
        wcur = jnp.where(wcur == mx, -jnp.inf, wcur)

    exps = [jnp.exp(v - vals[0]) for v in vals]
    inv = 1.0 / sum(exps)

    row_iota = jax.lax.broadcasted_iota(jnp.int32, (_L, _L), 0)
    l_iota = jax.lax.broadcasted_iota(jnp.int32, (_L, _L), 1)
    Wm = jnp.where(row_iota // _N == l_iota // _M, W, -jnp.inf)
    for kk in range(_K):
        Pk = jnp.where(Wm == vals[kk], exps[kk] * inv, 0.0)
        Gk = _dot(Pk, c)
        Ok = Gk * f
        for b in range(_BB):
            out_ref[pl.ds(b, 1), pl.Slice(kk, _N, _K), :] = (
                Ok[_N * b:_N * (b + 1), :].reshape(1, _N, _D))


def kernel(featureVec, contextVec, Wq, Wk):
    return pl.pallas_call(
        _attn_kernel,
        grid=(_B // _BB,),
        in_specs=[
            pl.BlockSpec((_BB, _N, _D), lambda i: (i, 0, 0)),
            pl.BlockSpec((_BB, _M, _D), lambda i: (i, 0, 0)),
            pl.BlockSpec((_D, _D), lambda i: (0, 0)),
            pl.BlockSpec((_D, _D), lambda i: (0, 0)),
        ],
        out_specs=pl.BlockSpec((_BB, _N * _K, _D), lambda i: (i, 0, 0)),
        out_shape=jax.ShapeDtypeStruct((_B, _N * _K, _D), jnp.float32),
    )(featureVec, contextVec, Wq, Wk)
